# baseline (speedup 1.0000x reference)
import jax
import jax.numpy as jnp
from jax.experimental import pallas as pl

_BLOCK_ROWS = 128


def _copy_block(in_ref, out_ref):
    out_ref[...] = in_ref[...]


def kernel(features):
    rows, cols = features.shape
    return pl.pallas_call(
        _copy_block,
        grid=(rows // _BLOCK_ROWS,),
        in_specs=[pl.BlockSpec((_BLOCK_ROWS, cols), lambda i: (i, 0))],
        out_specs=pl.BlockSpec((_BLOCK_ROWS, cols), lambda i: (i, 0)),
        out_shape=jax.ShapeDtypeStruct((rows, cols), features.dtype),
    )(features)


# speedup vs baseline: 1.1588x; 1.1588x over previous
import jax
import jax.numpy as jnp
from jax.experimental import pallas as pl

_BLOCK_COLS = 16384


def _copy_block(in_ref, out_ref):
    out_ref[...] = in_ref[...]


def kernel(features):
    rows, cols = features.shape
    return pl.pallas_call(
        _copy_block,
        grid=(cols // _BLOCK_COLS,),
        in_specs=[pl.BlockSpec((rows, _BLOCK_COLS), lambda i: (0, i))],
        out_specs=pl.BlockSpec((rows, _BLOCK_COLS), lambda i: (0, i)),
        out_shape=jax.ShapeDtypeStruct((rows, cols), features.dtype),
    )(features)
